# baseline (speedup 1.0000x reference)
import functools

import jax
import jax.numpy as jnp
from jax.experimental import pallas as pl
from jax.experimental.pallas import tpu as pltpu

_B, _N, _S, _K = 2, 8192, 2048, 32
_CIN, _COUT = 64, 64
_LEAKY = 0.1
_EPS = 1e-5


def _lk(x):
    return jnp.where(x >= 0, x, _LEAKY * x)


_FPS_R, _FPS_C = 64, 128


def _fps_body(x_ref, idx_ref, nxyz_ref):
    x = x_ref[0, 0]
    y = x_ref[0, 1]
    z = x_ref[0, 2]
    flat = (jax.lax.broadcasted_iota(jnp.int32, (_FPS_R, _FPS_C), 0) * _FPS_C
            + jax.lax.broadcasted_iota(jnp.int32, (_FPS_R, _FPS_C), 1))

    def body(i, carry):
        dists, far = carry
        idx_ref[0, 0, i] = far
        mask = flat == far
        cx = jnp.sum(jnp.where(mask, x, 0.0))
        cy = jnp.sum(jnp.where(mask, y, 0.0))
        cz = jnp.sum(jnp.where(mask, z, 0.0))
        nxyz_ref[0, 0, i] = cx
        nxyz_ref[0, 1, i] = cy
        nxyz_ref[0, 2, i] = cz
        dx = x - cx
        dy = y - cy
        dz = z - cz
        d = dx * dx + dy * dy + dz * dz
        dists = jnp.minimum(dists, d)
        m = jnp.max(dists)
        far = jnp.min(jnp.where(dists == m, flat, jnp.int32(_N)))
        return (dists, far)

    dists0 = jnp.full((_FPS_R, _FPS_C), 1e10, jnp.float32)
    jax.lax.fori_loop(0, _S, body, (dists0, jnp.int32(0)))


def _fps_pallas(xyz):
    xr = xyz.reshape(_B, 3, _FPS_R, _FPS_C)
    idx, nxyz = pl.pallas_call(
        _fps_body,
        grid=(_B,),
        in_specs=[pl.BlockSpec((1, 3, _FPS_R, _FPS_C), lambda b: (b, 0, 0, 0))],
        out_specs=[
            pl.BlockSpec((1, 1, _S), lambda b: (b, 0, 0), memory_space=pltpu.SMEM),
            pl.BlockSpec((1, 3, _S), lambda b: (b, 0, 0), memory_space=pltpu.SMEM),
        ],
        out_shape=[
            jax.ShapeDtypeStruct((_B, 1, _S), jnp.int32),
            jax.ShapeDtypeStruct((_B, 3, _S), jnp.float32),
        ],
    )(xr)
    return idx.reshape(_B, _S), nxyz


_KNN_ROWS = 8
_NCHUNK = _N // 128


def _batcher_pairs(n):
    pairs = []
    p = 1
    while p < n:
        k = p
        while k >= 1:
            for j in range(k % p, n - k, 2 * k):
                for i in range(0, min(k, n - j - k)):
                    if (i + j) // (2 * p) == (i + j + k) // (2 * p):
                        pairs.append((i + j, i + j + k))
            k //= 2
        p *= 2
    return pairs


def _pruned_net(n, keep):
    needed = set(range(keep))
    kept = []
    for (i, j) in reversed(_batcher_pairs(n)):
        if i in needed or j in needed:
            kept.append((i, j))
            needed.add(i)
            needed.add(j)
    kept.reverse()
    return kept


_KNN_NET = _pruned_net(_NCHUNK, _K)


def _knn_body(c_ref, x_ref, idx_ref):
    rows = _KNN_ROWS
    cb = c_ref[0]
    xb = x_ref[0]
    mm = jnp.dot(cb, xb, preferred_element_type=jnp.float32)
    cn = jnp.sum(cb * cb, axis=1, keepdims=True)
    xn = jnp.sum(xb * xb, axis=0, keepdims=True)
    d = -2.0 * mm
    d = d + cn
    d = d + xn

    keys = [d[:, 128 * c:128 * (c + 1)] for c in range(_NCHUNK)]
    lane = jax.lax.broadcasted_iota(jnp.int32, (rows, 128), 1)
    pay = [lane + 128 * c for c in range(_NCHUNK)]
    for (i, j) in _KNN_NET:
        a, b = keys[i], keys[j]
        m = a <= b
        keys[i] = jnp.minimum(a, b)
        keys[j] = jnp.maximum(a, b)
        pi, pj = pay[i], pay[j]
        pay[i] = jnp.where(m, pi, pj)
        pay[j] = jnp.where(m, pj, pi)

    F = keys[0]
    FI = pay[0]
    ptr = jnp.zeros((rows, 128), jnp.int32)
    lane_k = jax.lax.broadcasted_iota(jnp.int32, (rows, _K), 1)
    out = jnp.zeros((rows, _K), jnp.int32)
    for kk in range(_K):
        m = jnp.min(F, axis=1, keepdims=True)
        sel = F == m
        lsel = jnp.min(jnp.where(sel, lane, _N), axis=1, keepdims=True)
        lmask = lane == lsel
        ei = jnp.min(jnp.where(lmask, FI, _N), axis=1, keepdims=True)
        out = jnp.where(lane_k == kk, ei, out)
        if kk < _K - 1:
            ptr = ptr + lmask.astype(jnp.int32)
            depth = kk + 1
            nk = keys[depth]
            ni = pay[depth]
            for r in range(depth - 1, 0, -1):
                selr = ptr == r
                nk = jnp.where(selr, keys[r], nk)
                ni = jnp.where(selr, pay[r], ni)
            F = jnp.where(lmask, nk, F)
            FI = jnp.where(lmask, ni, FI)
    idx_ref[0] = out


def _knn_pallas(new_xyz, xyz):
    return pl.pallas_call(
        _knn_body,
        grid=(_B, _S // _KNN_ROWS),
        in_specs=[
            pl.BlockSpec((1, _KNN_ROWS, 3), lambda b, s: (b, s, 0)),
            pl.BlockSpec((1, 3, _N), lambda b, s: (b, 0, 0)),
        ],
        out_specs=pl.BlockSpec((1, _KNN_ROWS, _K), lambda b, s: (b, s, 0)),
        out_shape=jax.ShapeDtypeStruct((_B, _S, _K), jnp.int32),
    )(new_xyz, xyz)


def _tail_kernel(feat_ref, w1_ref, b1_ref, g1_ref, be1_ref, out_ref):
    f = feat_ref[...]
    w = w1_ref[...]
    y = jnp.dot(w, f, preferred_element_type=jnp.float32) + b1_ref[...][:, None]
    out_ref[...] = y


def _conv1_pallas(features):
    def one(f, w1, b1):
        return pl.pallas_call(
            lambda fr, wr, br, orr: orr.__setitem__(
                (...,), jnp.dot(wr[...], fr[...],
                                preferred_element_type=jnp.float32)
                + br[...][:, None]),
            out_shape=jax.ShapeDtypeStruct((_COUT, _S), jnp.float32),
        )(f, w1, b1)
    return one


def kernel(xyz, points, W0, b0, g0, be0, Wl, bl, gl, bel, Ws, W1, b1, g1, be1):
    xyz_t = xyz.transpose(0, 2, 1)
    pts_t = points.transpose(0, 2, 1)
    fps_idx, nxyz_cs = _fps_pallas(xyz)
    new_xyz = nxyz_cs.transpose(0, 2, 1)

    idx = _knn_pallas(new_xyz, xyz)

    grouped_xyz = jax.vmap(lambda p, i: p[i])(xyz_t, idx)
    grouped_norm = grouped_xyz - new_xyz[:, :, None, :]
    grouped_pts = jax.vmap(lambda p, i: p[i])(pts_t, idx)

    def bn(x, g, be):
        m = jnp.mean(x, axis=(0, 2, 3), keepdims=True)
        v = jnp.mean((x - m) ** 2, axis=(0, 2, 3), keepdims=True)
        return g[None, :, None, None] * (x - m) / jnp.sqrt(v + _EPS) + be[None, :, None, None]

    def conv(x, W, b=None):
        y = jnp.einsum('bchw,oc->bohw', x, W)
        if b is not None:
            y = y + b[None, :, None, None]
        return y

    new_points = _lk(bn(conv(grouped_pts.transpose(0, 3, 1, 2), W0, b0), g0, be0))
    gx = grouped_xyz.transpose(0, 3, 1, 2)
    gn = grouped_norm.transpose(0, 3, 1, 2)
    ext = jnp.broadcast_to(new_xyz.transpose(0, 2, 1)[:, :, :, None], (_B, 3, _S, _K))
    concat = jnp.concatenate([ext, gx, gn], axis=1)
    lse = _lk(bn(conv(concat, Wl, bl), gl, bel))
    lse1 = jnp.concatenate([lse, new_points], axis=1)
    scores = jax.nn.softmax(_lk(conv(lse1, Ws)), axis=-1)
    features = jnp.sum(scores * lse1, axis=-1)

    pre = jax.vmap(lambda f: pl.pallas_call(
        _tail_kernel,
        out_shape=jax.ShapeDtypeStruct((_COUT, _S), jnp.float32),
    )(f, W1, b1, g1, be1))(features)

    m = jnp.mean(pre, axis=(0, 2), keepdims=True)
    v = jnp.mean((pre - m) ** 2, axis=(0, 2), keepdims=True)
    out_points = _lk(g1[None, :, None] * (pre - m) / jnp.sqrt(v + _EPS) + be1[None, :, None])
    return (new_xyz.transpose(0, 2, 1), out_points, fps_idx)


# speedup vs baseline: 1.1723x; 1.1723x over previous
import functools

import jax
import jax.numpy as jnp
from jax.experimental import pallas as pl
from jax.experimental.pallas import tpu as pltpu

_B, _N, _S, _K = 2, 8192, 2048, 32
_CIN, _COUT = 64, 64
_LEAKY = 0.1
_EPS = 1e-5


def _lk(x):
    return jnp.where(x >= 0, x, _LEAKY * x)


_FPS_R, _FPS_C = 64, 128


def _fps_body(x_ref, idx_ref, nxyz_ref):
    x = x_ref[0, 0]
    y = x_ref[0, 1]
    z = x_ref[0, 2]
    flat = (jax.lax.broadcasted_iota(jnp.int32, (_FPS_R, _FPS_C), 0) * _FPS_C
            + jax.lax.broadcasted_iota(jnp.int32, (_FPS_R, _FPS_C), 1))

    def body(i, carry):
        dists, far = carry
        idx_ref[0, 0, i] = far
        mask = flat == far
        cx = jnp.sum(jnp.where(mask, x, 0.0))
        cy = jnp.sum(jnp.where(mask, y, 0.0))
        cz = jnp.sum(jnp.where(mask, z, 0.0))
        nxyz_ref[0, 0, i] = cx
        nxyz_ref[0, 1, i] = cy
        nxyz_ref[0, 2, i] = cz
        dx = x - cx
        dy = y - cy
        dz = z - cz
        d = dx * dx + dy * dy + dz * dz
        dists = jnp.minimum(dists, d)
        m = jnp.max(dists)
        far = jnp.min(jnp.where(dists == m, flat, jnp.int32(_N)))
        return (dists, far)

    dists0 = jnp.full((_FPS_R, _FPS_C), 1e10, jnp.float32)
    jax.lax.fori_loop(0, _S, body, (dists0, jnp.int32(0)))


def _fps_pallas(xyz):
    xr = xyz.reshape(_B, 3, _FPS_R, _FPS_C)
    idx, nxyz = pl.pallas_call(
        _fps_body,
        grid=(_B,),
        in_specs=[pl.BlockSpec((1, 3, _FPS_R, _FPS_C), lambda b: (b, 0, 0, 0))],
        out_specs=[
            pl.BlockSpec((1, 1, _S), lambda b: (b, 0, 0), memory_space=pltpu.SMEM),
            pl.BlockSpec((1, 3, _S), lambda b: (b, 0, 0), memory_space=pltpu.SMEM),
        ],
        out_shape=[
            jax.ShapeDtypeStruct((_B, 1, _S), jnp.int32),
            jax.ShapeDtypeStruct((_B, 3, _S), jnp.float32),
        ],
    )(xr)
    return idx.reshape(_B, _S), nxyz


_KNN_ROWS = 8
_NCHUNK = _N // 128


def _batcher_pairs(n):
    pairs = []
    p = 1
    while p < n:
        k = p
        while k >= 1:
            for j in range(k % p, n - k, 2 * k):
                for i in range(0, min(k, n - j - k)):
                    if (i + j) // (2 * p) == (i + j + k) // (2 * p):
                        pairs.append((i + j, i + j + k))
            k //= 2
        p *= 2
    return pairs


def _pruned_net(n, keep):
    needed = set(range(keep))
    kept = []
    for (i, j) in reversed(_batcher_pairs(n)):
        if i in needed or j in needed:
            kept.append((i, j))
            needed.add(i)
            needed.add(j)
    kept.reverse()
    return kept


_KNN_NET = _pruned_net(_NCHUNK, _K)


def _knn_body(c_ref, x_ref, idx_ref):
    rows = _KNN_ROWS
    cb = c_ref[0]
    xb = x_ref[0]
    mm = jnp.dot(cb, xb, preferred_element_type=jnp.float32)
    cn = jnp.sum(cb * cb, axis=1, keepdims=True)
    xn = jnp.sum(xb * xb, axis=0, keepdims=True)
    d = -2.0 * mm
    d = d + cn
    d = d + xn

    keys = [d[:, 128 * c:128 * (c + 1)] for c in range(_NCHUNK)]
    lane = jax.lax.broadcasted_iota(jnp.int32, (rows, 128), 1)
    pay = [lane + 128 * c for c in range(_NCHUNK)]
    for (i, j) in _KNN_NET:
        a, b = keys[i], keys[j]
        m = a <= b
        keys[i] = jnp.minimum(a, b)
        keys[j] = jnp.maximum(a, b)
        pi, pj = pay[i], pay[j]
        pay[i] = jnp.where(m, pi, pj)
        pay[j] = jnp.where(m, pj, pi)

    F = keys[0]
    FI = pay[0]
    ptr = jnp.zeros((rows, 128), jnp.int32)
    lane_k = jax.lax.broadcasted_iota(jnp.int32, (rows, _K), 1)
    out = jnp.zeros((rows, _K), jnp.int32)
    for kk in range(_K):
        m = jnp.min(F, axis=1, keepdims=True)
        sel = F == m
        lsel = jnp.min(jnp.where(sel, lane, _N), axis=1, keepdims=True)
        lmask = lane == lsel
        ei = jnp.min(jnp.where(lmask, FI, _N), axis=1, keepdims=True)
        out = jnp.where(lane_k == kk, ei, out)
        if kk < _K - 1:
            ptr = ptr + lmask.astype(jnp.int32)
            depth = kk + 1
            nk = keys[depth]
            ni = pay[depth]
            for r in range(depth - 1, 0, -1):
                selr = ptr == r
                nk = jnp.where(selr, keys[r], nk)
                ni = jnp.where(selr, pay[r], ni)
            F = jnp.where(lmask, nk, F)
            FI = jnp.where(lmask, ni, FI)
    idx_ref[0] = out


def _knn_pallas(new_xyz, xyz):
    return pl.pallas_call(
        _knn_body,
        grid=(_B, _S // _KNN_ROWS),
        in_specs=[
            pl.BlockSpec((1, _KNN_ROWS, 3), lambda b, s: (b, s, 0)),
            pl.BlockSpec((1, 3, _N), lambda b, s: (b, 0, 0)),
        ],
        out_specs=pl.BlockSpec((1, _KNN_ROWS, _K), lambda b, s: (b, s, 0)),
        out_shape=jax.ShapeDtypeStruct((_B, _S, _K), jnp.int32),
    )(new_xyz, xyz)


_TROWS = 256


def _gmat_kernel(in_ref, w_ref, out_ref):
    out_ref[0] = jnp.dot(in_ref[0], w_ref[...],
                         preferred_element_type=jnp.float32, precision=jax.lax.Precision.HIGHEST)


def _stats_kernel(g_ref, c_ref, wm_ref, out_ref):
    step = pl.program_id(0)
    c64 = jnp.dot(c_ref[...], wm_ref[...], preferred_element_type=jnp.float32, precision=jax.lax.Precision.HIGHEST)
    x = g_ref[...] + c64[:, None, :]
    s1 = jnp.sum(x, axis=(0, 1))[None, :]
    s2 = jnp.sum(x * x, axis=(0, 1))[None, :]
    acc = jnp.concatenate([s1, s2], axis=0)

    @pl.when(step == 0)
    def _():
        out_ref[...] = jnp.zeros_like(out_ref)

    out_ref[...] += acc


def _main_kernel(g_ref, c_ref, wm_ref, sc_ref, sh_ref, wst_ref, w1t_ref,
                 f1_ref, st_ref):
    step = pl.program_id(0)
    c64 = jnp.dot(c_ref[...], wm_ref[...], preferred_element_type=jnp.float32, precision=jax.lax.Precision.HIGHEST)
    x = g_ref[...] + c64[:, None, :]
    lse1 = x * sc_ref[0][None, None, :] + sh_ref[0][None, None, :]
    lse1 = jnp.where(lse1 >= 0, lse1, _LEAKY * lse1)
    l2 = lse1.reshape(_TROWS * _K, _COUT)
    z = jnp.dot(l2, wst_ref[...], preferred_element_type=jnp.float32, precision=jax.lax.Precision.HIGHEST)
    z = jnp.where(z >= 0, z, _LEAKY * z).reshape(_TROWS, _K, _COUT)
    zm = jnp.max(z, axis=1, keepdims=True)
    e = jnp.exp(z - zm)
    sc = e / jnp.sum(e, axis=1, keepdims=True)
    feat = jnp.sum(sc * lse1, axis=1)
    f1 = jnp.dot(feat, w1t_ref[...], preferred_element_type=jnp.float32, precision=jax.lax.Precision.HIGHEST)
    f1_ref[...] = f1
    s1 = jnp.sum(f1, axis=0)[None, :]
    s2 = jnp.sum(f1 * f1, axis=0)[None, :]
    acc = jnp.concatenate([s1, s2], axis=0)

    @pl.when(step == 0)
    def _():
        st_ref[...] = jnp.zeros_like(st_ref)

    st_ref[...] += acc


def _final_kernel(f1_ref, sc_ref, sh_ref, out_ref):
    y = f1_ref[...] * sc_ref[0][None, :] + sh_ref[0][None, :]
    out_ref[...] = jnp.where(y >= 0, y, _LEAKY * y)


def kernel(xyz, points, W0, b0, g0, be0, Wl, bl, gl, bel, Ws, W1, b1, g1, be1):
    xyz_t = xyz.transpose(0, 2, 1)
    pts_t = points.transpose(0, 2, 1)
    fps_idx, nxyz_cs = _fps_pallas(xyz)
    new_xyz = nxyz_cs.transpose(0, 2, 1)

    idx = _knn_pallas(new_xyz, xyz)

    half = _COUT // 2
    Wg = Wl[:, 3:6] + Wl[:, 6:9]
    Wm = Wl[:, 0:3] - Wl[:, 6:9]
    Wcomb = jnp.zeros((3 + _CIN, _COUT), jnp.float32)
    Wcomb = Wcomb.at[0:3, 0:half].set(Wg.T)
    Wcomb = Wcomb.at[3:, half:].set(W0.T)
    in2 = jnp.concatenate([xyz_t, pts_t], axis=-1)
    G = pl.pallas_call(
        _gmat_kernel,
        grid=(_B,),
        in_specs=[pl.BlockSpec((1, _N, 3 + _CIN), lambda b: (b, 0, 0)),
                  pl.BlockSpec((3 + _CIN, _COUT), lambda b: (0, 0))],
        out_specs=pl.BlockSpec((1, _N, _COUT), lambda b: (b, 0, 0)),
        out_shape=jax.ShapeDtypeStruct((_B, _N, _COUT), jnp.float32),
    )(in2, Wcomb)

    idx_flat = idx.reshape(_B, _S * _K)
    Gg = jax.vmap(lambda g, i: g[i])(G, idx_flat)
    Gg = Gg.reshape(_B * _S, _K, _COUT)

    nxyz_flat = new_xyz.reshape(_B * _S, 3)
    Wm64 = jnp.zeros((3, _COUT), jnp.float32).at[:, 0:half].set(Wm.T)

    nsteps = (_B * _S) // _TROWS
    row_spec = pl.BlockSpec((_TROWS, _K, _COUT), lambda s: (s, 0, 0))
    c_spec = pl.BlockSpec((_TROWS, 3), lambda s: (s, 0))
    wm_spec = pl.BlockSpec((3, _COUT), lambda s: (0, 0))
    acc_spec = pl.BlockSpec((2, _COUT), lambda s: (0, 0))
    vec_spec = pl.BlockSpec((1, _COUT), lambda s: (0, 0))
    w64_spec = pl.BlockSpec((_COUT, _COUT), lambda s: (0, 0))

    sums = pl.pallas_call(
        _stats_kernel,
        grid=(nsteps,),
        in_specs=[row_spec, c_spec, wm_spec],
        out_specs=acc_spec,
        out_shape=jax.ShapeDtypeStruct((2, _COUT), jnp.float32),
    )(Gg, nxyz_flat, Wm64)

    n = float(_B * _S * _K)
    gf = jnp.concatenate([gl, g0])
    bef = jnp.concatenate([bel, be0])
    mean_x = sums[0] / n
    var = sums[1] / n - mean_x * mean_x
    scale64 = gf / jnp.sqrt(var + _EPS)
    shift64 = bef - mean_x * scale64

    f1, st = pl.pallas_call(
        _main_kernel,
        grid=(nsteps,),
        in_specs=[row_spec, c_spec, wm_spec, vec_spec, vec_spec,
                  w64_spec, w64_spec],
        out_specs=[pl.BlockSpec((_TROWS, _COUT), lambda s: (s, 0)), acc_spec],
        out_shape=[jax.ShapeDtypeStruct((_B * _S, _COUT), jnp.float32),
                   jax.ShapeDtypeStruct((2, _COUT), jnp.float32)],
    )(Gg, nxyz_flat, Wm64, scale64[None], shift64[None], Ws.T, W1.T)

    n1 = float(_B * _S)
    mean1 = st[0] / n1
    var1 = st[1] / n1 - mean1 * mean1
    scale1 = g1 / jnp.sqrt(var1 + _EPS)
    shift1 = be1 - mean1 * scale1

    outp = pl.pallas_call(
        _final_kernel,
        grid=(_B,),
        in_specs=[pl.BlockSpec((_S, _COUT), lambda b: (b, 0)),
                  pl.BlockSpec((1, _COUT), lambda b: (0, 0)),
                  pl.BlockSpec((1, _COUT), lambda b: (0, 0))],
        out_specs=pl.BlockSpec((_S, _COUT), lambda b: (b, 0)),
        out_shape=jax.ShapeDtypeStruct((_B * _S, _COUT), jnp.float32),
    )(f1, scale1[None], shift1[None])

    out_points = outp.reshape(_B, _S, _COUT).transpose(0, 2, 1)
    return (nxyz_cs, out_points, fps_idx)


# speedup vs baseline: 1.5735x; 1.3422x over previous
import functools

import jax
import jax.numpy as jnp
from jax.experimental import pallas as pl
from jax.experimental.pallas import tpu as pltpu
from jax.experimental.pallas import tpu_sc as plsc

_B, _N, _S, _K = 2, 8192, 2048, 32
_CIN, _COUT = 64, 64
_LEAKY = 0.1
_EPS = 1e-5


def _lk(x):
    return jnp.where(x >= 0, x, _LEAKY * x)


_FPS_R, _FPS_C = 64, 128


def _fps_body(x_ref, idx_ref, nxyz_ref):
    x = x_ref[0, 0]
    y = x_ref[0, 1]
    z = x_ref[0, 2]
    flat = (jax.lax.broadcasted_iota(jnp.int32, (_FPS_R, _FPS_C), 0) * _FPS_C
            + jax.lax.broadcasted_iota(jnp.int32, (_FPS_R, _FPS_C), 1))

    def body(i, carry):
        dists, far = carry
        idx_ref[0, 0, i] = far
        mask = flat == far
        cx = jnp.sum(jnp.where(mask, x, 0.0))
        cy = jnp.sum(jnp.where(mask, y, 0.0))
        cz = jnp.sum(jnp.where(mask, z, 0.0))
        nxyz_ref[0, 0, i] = cx
        nxyz_ref[0, 1, i] = cy
        nxyz_ref[0, 2, i] = cz
        dx = x - cx
        dy = y - cy
        dz = z - cz
        d = dx * dx + dy * dy + dz * dz
        dists = jnp.minimum(dists, d)
        m = jnp.max(dists)
        far = jnp.min(jnp.where(dists == m, flat, jnp.int32(_N)))
        return (dists, far)

    dists0 = jnp.full((_FPS_R, _FPS_C), 1e10, jnp.float32)
    jax.lax.fori_loop(0, _S, body, (dists0, jnp.int32(0)))


def _fps_pallas(xyz):
    xr = xyz.reshape(_B, 3, _FPS_R, _FPS_C)
    idx, nxyz = pl.pallas_call(
        _fps_body,
        grid=(_B,),
        in_specs=[pl.BlockSpec((1, 3, _FPS_R, _FPS_C), lambda b: (b, 0, 0, 0))],
        out_specs=[
            pl.BlockSpec((1, 1, _S), lambda b: (b, 0, 0), memory_space=pltpu.SMEM),
            pl.BlockSpec((1, 3, _S), lambda b: (b, 0, 0), memory_space=pltpu.SMEM),
        ],
        out_shape=[
            jax.ShapeDtypeStruct((_B, 1, _S), jnp.int32),
            jax.ShapeDtypeStruct((_B, 3, _S), jnp.float32),
        ],
    )(xr)
    return idx.reshape(_B, _S), nxyz


_KNN_ROWS = 8
_NCHUNK = _N // 128


def _batcher_pairs(n):
    pairs = []
    p = 1
    while p < n:
        k = p
        while k >= 1:
            for j in range(k % p, n - k, 2 * k):
                for i in range(0, min(k, n - j - k)):
                    if (i + j) // (2 * p) == (i + j + k) // (2 * p):
                        pairs.append((i + j, i + j + k))
            k //= 2
        p *= 2
    return pairs


def _pruned_net(n, keep):
    needed = set(range(keep))
    kept = []
    for (i, j) in reversed(_batcher_pairs(n)):
        if i in needed or j in needed:
            kept.append((i, j))
            needed.add(i)
            needed.add(j)
    kept.reverse()
    return kept


_KNN_NET = _pruned_net(_NCHUNK, _K)


def _knn_body(c_ref, x_ref, idx_ref):
    rows = _KNN_ROWS
    cb = c_ref[0]
    xb = x_ref[0]
    mm = jnp.dot(cb, xb, preferred_element_type=jnp.float32)
    cn = jnp.sum(cb * cb, axis=1, keepdims=True)
    xn = jnp.sum(xb * xb, axis=0, keepdims=True)
    d = -2.0 * mm
    d = d + cn
    d = d + xn

    keys = [d[:, 128 * c:128 * (c + 1)] for c in range(_NCHUNK)]
    lane = jax.lax.broadcasted_iota(jnp.int32, (rows, 128), 1)
    pay = [lane + 128 * c for c in range(_NCHUNK)]
    for (i, j) in _KNN_NET:
        a, b = keys[i], keys[j]
        m = a <= b
        keys[i] = jnp.minimum(a, b)
        keys[j] = jnp.maximum(a, b)
        pi, pj = pay[i], pay[j]
        pay[i] = jnp.where(m, pi, pj)
        pay[j] = jnp.where(m, pj, pi)

    F = keys[0]
    FI = pay[0]
    ptr = jnp.zeros((rows, 128), jnp.int32)
    lane_k = jax.lax.broadcasted_iota(jnp.int32, (rows, _K), 1)
    out = jnp.zeros((rows, _K), jnp.int32)
    for kk in range(_K):
        m = jnp.min(F, axis=1, keepdims=True)
        sel = F == m
        lsel = jnp.min(jnp.where(sel, lane, _N), axis=1, keepdims=True)
        lmask = lane == lsel
        ei = jnp.min(jnp.where(lmask, FI, _N), axis=1, keepdims=True)
        out = jnp.where(lane_k == kk, ei, out)
        if kk < _K - 1:
            ptr = ptr + lmask.astype(jnp.int32)
            depth = kk + 1
            nk = keys[depth]
            ni = pay[depth]
            for r in range(depth - 1, 0, -1):
                selr = ptr == r
                nk = jnp.where(selr, keys[r], nk)
                ni = jnp.where(selr, pay[r], ni)
            F = jnp.where(lmask, nk, F)
            FI = jnp.where(lmask, ni, FI)
    idx_ref[0] = out


def _knn_pallas(new_xyz, xyz):
    return pl.pallas_call(
        _knn_body,
        grid=(_B, _S // _KNN_ROWS),
        in_specs=[
            pl.BlockSpec((1, _KNN_ROWS, 3), lambda b, s: (b, s, 0)),
            pl.BlockSpec((1, 3, _N), lambda b, s: (b, 0, 0)),
        ],
        out_specs=pl.BlockSpec((1, _KNN_ROWS, _K), lambda b, s: (b, s, 0)),
        out_shape=jax.ShapeDtypeStruct((_B, _S, _K), jnp.int32),
    )(new_xyz, xyz)


_TROWS = 256

_SC_NW = 32
_SC_CHUNK = 128
_SC_PER_W = (_B * _S * _K) // _SC_NW


def _sc_gather(table, idxs):
    mesh = plsc.VectorSubcoreMesh(core_axis_name="c", subcore_axis_name="s")

    @functools.partial(
        pl.kernel, mesh=mesh,
        out_type=jax.ShapeDtypeStruct((_B * _S * _K, 128), jnp.float32),
        scratch_types=[
            pltpu.VMEM((_SC_PER_W,), jnp.int32),
            pltpu.VMEM((_SC_CHUNK, 128), jnp.float32),
            pltpu.SemaphoreType.DMA,
        ],
    )
    def k(table_hbm, idx_hbm, out_hbm, idx_v, rows_v, sem):
        wid = jax.lax.axis_index("s") * 2 + jax.lax.axis_index("c")
        base = wid * _SC_PER_W
        pltpu.sync_copy(idx_hbm.at[pl.ds(base, _SC_PER_W)], idx_v)

        def body(j, carry):
            off = j * _SC_CHUNK
            pltpu.async_copy(
                table_hbm.at[idx_v.at[pl.ds(off, _SC_CHUNK)]], rows_v, sem
            ).wait()
            pltpu.sync_copy(rows_v, out_hbm.at[pl.ds(base + off, _SC_CHUNK)])
            return carry

        jax.lax.fori_loop(0, _SC_PER_W // _SC_CHUNK, body, 0)

    return k(table, idxs)


def _gmat_kernel(in_ref, w_ref, out_ref):
    out_ref[0] = jnp.dot(in_ref[0], w_ref[...],
                         preferred_element_type=jnp.float32, precision=jax.lax.Precision.HIGHEST)


def _stats_kernel(g_ref, c_ref, wm_ref, out_ref):
    step = pl.program_id(0)
    c64 = jnp.dot(c_ref[...], wm_ref[...], preferred_element_type=jnp.float32, precision=jax.lax.Precision.HIGHEST)
    x = g_ref[...][:, :, :_COUT] + c64[:, None, :]
    s1 = jnp.sum(x, axis=(0, 1))[None, :]
    s2 = jnp.sum(x * x, axis=(0, 1))[None, :]
    acc = jnp.concatenate([s1, s2], axis=0)

    @pl.when(step == 0)
    def _():
        out_ref[...] = jnp.zeros_like(out_ref)

    out_ref[...] += acc


def _main_kernel(g_ref, c_ref, wm_ref, sc_ref, sh_ref, wst_ref, w1t_ref,
                 f1_ref, st_ref):
    step = pl.program_id(0)
    c64 = jnp.dot(c_ref[...], wm_ref[...], preferred_element_type=jnp.float32, precision=jax.lax.Precision.HIGHEST)
    x = g_ref[...][:, :, :_COUT] + c64[:, None, :]
    lse1 = x * sc_ref[0][None, None, :] + sh_ref[0][None, None, :]
    lse1 = jnp.where(lse1 >= 0, lse1, _LEAKY * lse1)
    l2 = lse1.reshape(_TROWS * _K, _COUT)
    z = jnp.dot(l2, wst_ref[...], preferred_element_type=jnp.float32, precision=jax.lax.Precision.HIGHEST)
    z = jnp.where(z >= 0, z, _LEAKY * z).reshape(_TROWS, _K, _COUT)
    zm = jnp.max(z, axis=1, keepdims=True)
    e = jnp.exp(z - zm)
    sc = e / jnp.sum(e, axis=1, keepdims=True)
    feat = jnp.sum(sc * lse1, axis=1)
    f1 = jnp.dot(feat, w1t_ref[...], preferred_element_type=jnp.float32, precision=jax.lax.Precision.HIGHEST)
    f1_ref[...] = f1
    s1 = jnp.sum(f1, axis=0)[None, :]
    s2 = jnp.sum(f1 * f1, axis=0)[None, :]
    acc = jnp.concatenate([s1, s2], axis=0)

    @pl.when(step == 0)
    def _():
        st_ref[...] = jnp.zeros_like(st_ref)

    st_ref[...] += acc


def _final_kernel(f1_ref, sc_ref, sh_ref, out_ref):
    y = f1_ref[...] * sc_ref[0][None, :] + sh_ref[0][None, :]
    out_ref[...] = jnp.where(y >= 0, y, _LEAKY * y)


def kernel(xyz, points, W0, b0, g0, be0, Wl, bl, gl, bel, Ws, W1, b1, g1, be1):
    xyz_t = xyz.transpose(0, 2, 1)
    pts_t = points.transpose(0, 2, 1)
    fps_idx, nxyz_cs = _fps_pallas(xyz)
    new_xyz = nxyz_cs.transpose(0, 2, 1)

    idx = _knn_pallas(new_xyz, xyz)

    half = _COUT // 2
    Wg = Wl[:, 3:6] + Wl[:, 6:9]
    Wm = Wl[:, 0:3] - Wl[:, 6:9]
    Wcomb = jnp.zeros((3 + _CIN, 128), jnp.float32)
    Wcomb = Wcomb.at[0:3, 0:half].set(Wg.T)
    Wcomb = Wcomb.at[3:, half:_COUT].set(W0.T)
    in2 = jnp.concatenate([xyz_t, pts_t], axis=-1)
    G = pl.pallas_call(
        _gmat_kernel,
        grid=(_B,),
        in_specs=[pl.BlockSpec((1, _N, 3 + _CIN), lambda b: (b, 0, 0)),
                  pl.BlockSpec((3 + _CIN, 128), lambda b: (0, 0))],
        out_specs=pl.BlockSpec((1, _N, 128), lambda b: (b, 0, 0)),
        out_shape=jax.ShapeDtypeStruct((_B, _N, 128), jnp.float32),
    )(in2, Wcomb)

    idx_glob = (idx + (jnp.arange(_B, dtype=jnp.int32) * _N)[:, None, None])
    Gg = _sc_gather(G.reshape(_B * _N, 128), idx_glob.reshape(-1))
    Gg = Gg.reshape(_B * _S, _K, 128)

    nxyz_flat = new_xyz.reshape(_B * _S, 3)
    Wm64 = jnp.zeros((3, _COUT), jnp.float32).at[:, 0:half].set(Wm.T)

    nsteps = (_B * _S) // _TROWS
    row_spec = pl.BlockSpec((_TROWS, _K, 128), lambda s: (s, 0, 0))
    c_spec = pl.BlockSpec((_TROWS, 3), lambda s: (s, 0))
    wm_spec = pl.BlockSpec((3, _COUT), lambda s: (0, 0))
    acc_spec = pl.BlockSpec((2, _COUT), lambda s: (0, 0))
    vec_spec = pl.BlockSpec((1, _COUT), lambda s: (0, 0))
    w64_spec = pl.BlockSpec((_COUT, _COUT), lambda s: (0, 0))

    sums = pl.pallas_call(
        _stats_kernel,
        grid=(nsteps,),
        in_specs=[row_spec, c_spec, wm_spec],
        out_specs=acc_spec,
        out_shape=jax.ShapeDtypeStruct((2, _COUT), jnp.float32),
    )(Gg, nxyz_flat, Wm64)

    n = float(_B * _S * _K)
    gf = jnp.concatenate([gl, g0])
    bef = jnp.concatenate([bel, be0])
    mean_x = sums[0] / n
    var = sums[1] / n - mean_x * mean_x
    scale64 = gf / jnp.sqrt(var + _EPS)
    shift64 = bef - mean_x * scale64

    f1, st = pl.pallas_call(
        _main_kernel,
        grid=(nsteps,),
        in_specs=[row_spec, c_spec, wm_spec, vec_spec, vec_spec,
                  w64_spec, w64_spec],
        out_specs=[pl.BlockSpec((_TROWS, _COUT), lambda s: (s, 0)), acc_spec],
        out_shape=[jax.ShapeDtypeStruct((_B * _S, _COUT), jnp.float32),
                   jax.ShapeDtypeStruct((2, _COUT), jnp.float32)],
    )(Gg, nxyz_flat, Wm64, scale64[None], shift64[None], Ws.T, W1.T)

    n1 = float(_B * _S)
    mean1 = st[0] / n1
    var1 = st[1] / n1 - mean1 * mean1
    scale1 = g1 / jnp.sqrt(var1 + _EPS)
    shift1 = be1 - mean1 * scale1

    outp = pl.pallas_call(
        _final_kernel,
        grid=(_B,),
        in_specs=[pl.BlockSpec((_S, _COUT), lambda b: (b, 0)),
                  pl.BlockSpec((1, _COUT), lambda b: (0, 0)),
                  pl.BlockSpec((1, _COUT), lambda b: (0, 0))],
        out_specs=pl.BlockSpec((_S, _COUT), lambda b: (b, 0)),
        out_shape=jax.ShapeDtypeStruct((_B * _S, _COUT), jnp.float32),
    )(f1, scale1[None], shift1[None])

    out_points = outp.reshape(_B, _S, _COUT).transpose(0, 2, 1)
    return (nxyz_cs, out_points, fps_idx)


# speedup vs baseline: 2.2883x; 1.4543x over previous
import functools

import jax
import jax.numpy as jnp
from jax.experimental import pallas as pl
from jax.experimental.pallas import tpu as pltpu
from jax.experimental.pallas import tpu_sc as plsc

_B, _N, _S, _K = 2, 8192, 2048, 32
_CIN, _COUT = 64, 64
_LEAKY = 0.1
_EPS = 1e-5


def _lk(x):
    return jnp.where(x >= 0, x, _LEAKY * x)


_FPS_R, _FPS_C = 64, 128


def _fps_body(x_ref, idx_ref, nxyz_ref):
    x = x_ref[0, 0]
    y = x_ref[0, 1]
    z = x_ref[0, 2]
    flat = (jax.lax.broadcasted_iota(jnp.int32, (_FPS_R, _FPS_C), 0) * _FPS_C
            + jax.lax.broadcasted_iota(jnp.int32, (_FPS_R, _FPS_C), 1))

    def body(i, carry):
        dists, far = carry
        idx_ref[0, 0, i] = far
        mask = flat == far
        cx = jnp.sum(jnp.where(mask, x, 0.0))
        cy = jnp.sum(jnp.where(mask, y, 0.0))
        cz = jnp.sum(jnp.where(mask, z, 0.0))
        nxyz_ref[0, 0, i] = cx
        nxyz_ref[0, 1, i] = cy
        nxyz_ref[0, 2, i] = cz
        dx = x - cx
        dy = y - cy
        dz = z - cz
        d = dx * dx + dy * dy + dz * dz
        dists = jnp.minimum(dists, d)
        m = jnp.max(dists)
        far = jnp.min(jnp.where(dists == m, flat, jnp.int32(_N)))
        return (dists, far)

    dists0 = jnp.full((_FPS_R, _FPS_C), 1e10, jnp.float32)
    jax.lax.fori_loop(0, _S, body, (dists0, jnp.int32(0)))


def _fps_pallas(xyz):
    xr = xyz.reshape(_B, 3, _FPS_R, _FPS_C)
    idx, nxyz = pl.pallas_call(
        _fps_body,
        grid=(_B,),
        in_specs=[pl.BlockSpec((1, 3, _FPS_R, _FPS_C), lambda b: (b, 0, 0, 0))],
        out_specs=[
            pl.BlockSpec((1, 1, _S), lambda b: (b, 0, 0), memory_space=pltpu.SMEM),
            pl.BlockSpec((1, 3, _S), lambda b: (b, 0, 0), memory_space=pltpu.SMEM),
        ],
        out_shape=[
            jax.ShapeDtypeStruct((_B, 1, _S), jnp.int32),
            jax.ShapeDtypeStruct((_B, 3, _S), jnp.float32),
        ],
    )(xr)
    return idx.reshape(_B, _S), nxyz


_KNN_ROWS = 16
_NCHUNK = _N // 128


def _batcher_pairs(n):
    pairs = []
    p = 1
    while p < n:
        k = p
        while k >= 1:
            for j in range(k % p, n - k, 2 * k):
                for i in range(0, min(k, n - j - k)):
                    if (i + j) // (2 * p) == (i + j + k) // (2 * p):
                        pairs.append((i + j, i + j + k))
            k //= 2
        p *= 2
    return pairs


def _pruned_net(n, keep):
    needed = set(range(keep))
    kept = []
    for (i, j) in reversed(_batcher_pairs(n)):
        if i in needed or j in needed:
            kept.append((i, j))
            needed.add(i)
            needed.add(j)
    kept.reverse()
    return kept


_KNN_NET = _pruned_net(_NCHUNK, _K)


def _knn_body(c_ref, x_ref, idx_ref):
    rows = _KNN_ROWS
    cb = c_ref[0]
    xb = x_ref[0]
    mm = jnp.dot(cb, xb, preferred_element_type=jnp.float32)
    cn = jnp.sum(cb * cb, axis=1, keepdims=True)
    xn = jnp.sum(xb * xb, axis=0, keepdims=True)
    d = -2.0 * mm
    d = d + cn
    d = d + xn

    keys = [d[:, 128 * c:128 * (c + 1)] for c in range(_NCHUNK)]
    lane = jax.lax.broadcasted_iota(jnp.int32, (rows, 128), 1)
    pay = [lane + 128 * c for c in range(_NCHUNK)]
    for (i, j) in _KNN_NET:
        a, b = keys[i], keys[j]
        m = a <= b
        keys[i] = jnp.minimum(a, b)
        keys[j] = jnp.maximum(a, b)
        pi, pj = pay[i], pay[j]
        pay[i] = jnp.where(m, pi, pj)
        pay[j] = jnp.where(m, pj, pi)

    F = keys[0]
    FI = pay[0]
    ptr = jnp.zeros((rows, 128), jnp.int32)
    lane_k = jax.lax.broadcasted_iota(jnp.int32, (rows, _K), 1)
    out = jnp.zeros((rows, _K), jnp.int32)
    for kk in range(_K):
        m = jnp.min(F, axis=1, keepdims=True)
        sel = F == m
        lsel = jnp.min(jnp.where(sel, lane, _N), axis=1, keepdims=True)
        lmask = lane == lsel
        ei = jnp.min(jnp.where(lmask, FI, _N), axis=1, keepdims=True)
        out = jnp.where(lane_k == kk, ei, out)
        if kk < _K - 1:
            ptr = ptr + lmask.astype(jnp.int32)
            depth = kk + 1
            nk = keys[depth]
            ni = pay[depth]
            for r in range(depth - 1, 0, -1):
                selr = ptr == r
                nk = jnp.where(selr, keys[r], nk)
                ni = jnp.where(selr, pay[r], ni)
            F = jnp.where(lmask, nk, F)
            FI = jnp.where(lmask, ni, FI)
    idx_ref[0] = out


def _knn_pallas(new_xyz, xyz):
    return pl.pallas_call(
        _knn_body,
        grid=(_B, _S // _KNN_ROWS),
        in_specs=[
            pl.BlockSpec((1, _KNN_ROWS, 3), lambda b, s: (b, s, 0)),
            pl.BlockSpec((1, 3, _N), lambda b, s: (b, 0, 0)),
        ],
        out_specs=pl.BlockSpec((1, _KNN_ROWS, _K), lambda b, s: (b, s, 0)),
        out_shape=jax.ShapeDtypeStruct((_B, _S, _K), jnp.int32),
    )(new_xyz, xyz)


_TROWS = 256

_SC_NW = 32
_SC_CHUNK = 128
_SC_PER_W = (_B * _S * _K) // _SC_NW


def _sc_gather(table, idxs):
    mesh = plsc.VectorSubcoreMesh(core_axis_name="c", subcore_axis_name="s")

    @functools.partial(
        pl.kernel, mesh=mesh,
        out_type=jax.ShapeDtypeStruct((_B * _S * _K, 128), jnp.float32),
        scratch_types=[
            pltpu.VMEM((_SC_PER_W,), jnp.int32),
            pltpu.VMEM((_SC_CHUNK, 128), jnp.float32),
            pltpu.SemaphoreType.DMA,
        ],
    )
    def k(table_hbm, idx_hbm, out_hbm, idx_v, rows_v, sem):
        wid = jax.lax.axis_index("s") * 2 + jax.lax.axis_index("c")
        base = wid * _SC_PER_W
        pltpu.sync_copy(idx_hbm.at[pl.ds(base, _SC_PER_W)], idx_v)

        def body(j, carry):
            off = j * _SC_CHUNK
            pltpu.async_copy(
                table_hbm.at[idx_v.at[pl.ds(off, _SC_CHUNK)]], rows_v, sem
            ).wait()
            pltpu.sync_copy(rows_v, out_hbm.at[pl.ds(base + off, _SC_CHUNK)])
            return carry

        jax.lax.fori_loop(0, _SC_PER_W // _SC_CHUNK, body, 0)

    return k(table, idxs)


def _gmat_kernel(in_ref, w_ref, out_ref):
    out_ref[0] = jnp.dot(in_ref[0], w_ref[...],
                         preferred_element_type=jnp.float32, precision=jax.lax.Precision.HIGHEST)


def _stats_kernel(g_ref, c_ref, wm_ref, out_ref):
    step = pl.program_id(0)
    c64 = jnp.dot(c_ref[...], wm_ref[...], preferred_element_type=jnp.float32, precision=jax.lax.Precision.HIGHEST)
    x = g_ref[...][:, :, :_COUT] + c64[:, None, :]
    s1 = jnp.sum(x, axis=(0, 1))[None, :]
    s2 = jnp.sum(x * x, axis=(0, 1))[None, :]
    acc = jnp.concatenate([s1, s2], axis=0)

    @pl.when(step == 0)
    def _():
        out_ref[...] = jnp.zeros_like(out_ref)

    out_ref[...] += acc


def _main_kernel(g_ref, c_ref, wm_ref, sc_ref, sh_ref, wst_ref, w1t_ref,
                 f1_ref, st_ref):
    step = pl.program_id(0)
    c64 = jnp.dot(c_ref[...], wm_ref[...], preferred_element_type=jnp.float32, precision=jax.lax.Precision.HIGHEST)
    x = g_ref[...][:, :, :_COUT] + c64[:, None, :]
    lse1 = x * sc_ref[0][None, None, :] + sh_ref[0][None, None, :]
    lse1 = jnp.where(lse1 >= 0, lse1, _LEAKY * lse1)
    l2 = lse1.reshape(_TROWS * _K, _COUT)
    z = jnp.dot(l2, wst_ref[...], preferred_element_type=jnp.float32, precision=jax.lax.Precision.HIGHEST)
    z = jnp.where(z >= 0, z, _LEAKY * z).reshape(_TROWS, _K, _COUT)
    zm = jnp.max(z, axis=1, keepdims=True)
    e = jnp.exp(z - zm)
    sc = e / jnp.sum(e, axis=1, keepdims=True)
    feat = jnp.sum(sc * lse1, axis=1)
    f1 = jnp.dot(feat, w1t_ref[...], preferred_element_type=jnp.float32, precision=jax.lax.Precision.HIGHEST)
    f1_ref[...] = f1
    s1 = jnp.sum(f1, axis=0)[None, :]
    s2 = jnp.sum(f1 * f1, axis=0)[None, :]
    acc = jnp.concatenate([s1, s2], axis=0)

    @pl.when(step == 0)
    def _():
        st_ref[...] = jnp.zeros_like(st_ref)

    st_ref[...] += acc


def _final_kernel(f1_ref, sc_ref, sh_ref, out_ref):
    y = f1_ref[...] * sc_ref[0][None, :] + sh_ref[0][None, :]
    out_ref[...] = jnp.where(y >= 0, y, _LEAKY * y)


def kernel(xyz, points, W0, b0, g0, be0, Wl, bl, gl, bel, Ws, W1, b1, g1, be1):
    xyz_t = xyz.transpose(0, 2, 1)
    pts_t = points.transpose(0, 2, 1)
    fps_idx, nxyz_cs = _fps_pallas(xyz)
    new_xyz = nxyz_cs.transpose(0, 2, 1)

    idx = _knn_pallas(new_xyz, xyz)

    half = _COUT // 2
    Wg = Wl[:, 3:6] + Wl[:, 6:9]
    Wm = Wl[:, 0:3] - Wl[:, 6:9]
    Wcomb = jnp.zeros((3 + _CIN, 128), jnp.float32)
    Wcomb = Wcomb.at[0:3, 0:half].set(Wg.T)
    Wcomb = Wcomb.at[3:, half:_COUT].set(W0.T)
    in2 = jnp.concatenate([xyz_t, pts_t], axis=-1)
    G = pl.pallas_call(
        _gmat_kernel,
        grid=(_B,),
        in_specs=[pl.BlockSpec((1, _N, 3 + _CIN), lambda b: (b, 0, 0)),
                  pl.BlockSpec((3 + _CIN, 128), lambda b: (0, 0))],
        out_specs=pl.BlockSpec((1, _N, 128), lambda b: (b, 0, 0)),
        out_shape=jax.ShapeDtypeStruct((_B, _N, 128), jnp.float32),
    )(in2, Wcomb)

    idx_glob = (idx + (jnp.arange(_B, dtype=jnp.int32) * _N)[:, None, None])
    Gg = _sc_gather(G.reshape(_B * _N, 128), idx_glob.reshape(-1))
    Gg = Gg.reshape(_B * _S, _K, 128)

    nxyz_flat = new_xyz.reshape(_B * _S, 3)
    Wm64 = jnp.zeros((3, _COUT), jnp.float32).at[:, 0:half].set(Wm.T)

    nsteps = (_B * _S) // _TROWS
    row_spec = pl.BlockSpec((_TROWS, _K, 128), lambda s: (s, 0, 0))
    c_spec = pl.BlockSpec((_TROWS, 3), lambda s: (s, 0))
    wm_spec = pl.BlockSpec((3, _COUT), lambda s: (0, 0))
    acc_spec = pl.BlockSpec((2, _COUT), lambda s: (0, 0))
    vec_spec = pl.BlockSpec((1, _COUT), lambda s: (0, 0))
    w64_spec = pl.BlockSpec((_COUT, _COUT), lambda s: (0, 0))

    sums = pl.pallas_call(
        _stats_kernel,
        grid=(nsteps,),
        in_specs=[row_spec, c_spec, wm_spec],
        out_specs=acc_spec,
        out_shape=jax.ShapeDtypeStruct((2, _COUT), jnp.float32),
    )(Gg, nxyz_flat, Wm64)

    n = float(_B * _S * _K)
    gf = jnp.concatenate([gl, g0])
    bef = jnp.concatenate([bel, be0])
    mean_x = sums[0] / n
    var = sums[1] / n - mean_x * mean_x
    scale64 = gf / jnp.sqrt(var + _EPS)
    shift64 = bef - mean_x * scale64

    f1, st = pl.pallas_call(
        _main_kernel,
        grid=(nsteps,),
        in_specs=[row_spec, c_spec, wm_spec, vec_spec, vec_spec,
                  w64_spec, w64_spec],
        out_specs=[pl.BlockSpec((_TROWS, _COUT), lambda s: (s, 0)), acc_spec],
        out_shape=[jax.ShapeDtypeStruct((_B * _S, _COUT), jnp.float32),
                   jax.ShapeDtypeStruct((2, _COUT), jnp.float32)],
    )(Gg, nxyz_flat, Wm64, scale64[None], shift64[None], Ws.T, W1.T)

    n1 = float(_B * _S)
    mean1 = st[0] / n1
    var1 = st[1] / n1 - mean1 * mean1
    scale1 = g1 / jnp.sqrt(var1 + _EPS)
    shift1 = be1 - mean1 * scale1

    outp = pl.pallas_call(
        _final_kernel,
        grid=(_B,),
        in_specs=[pl.BlockSpec((_S, _COUT), lambda b: (b, 0)),
                  pl.BlockSpec((1, _COUT), lambda b: (0, 0)),
                  pl.BlockSpec((1, _COUT), lambda b: (0, 0))],
        out_specs=pl.BlockSpec((_S, _COUT), lambda b: (b, 0)),
        out_shape=jax.ShapeDtypeStruct((_B * _S, _COUT), jnp.float32),
    )(f1, scale1[None], shift1[None])

    out_points = outp.reshape(_B, _S, _COUT).transpose(0, 2, 1)
    return (nxyz_cs, out_points, fps_idx)


# speedup vs baseline: 2.9996x; 1.3108x over previous
import functools

import jax
import jax.numpy as jnp
from jax.experimental import pallas as pl
from jax.experimental.pallas import tpu as pltpu
from jax.experimental.pallas import tpu_sc as plsc

_B, _N, _S, _K = 2, 8192, 2048, 32
_CIN, _COUT = 64, 64
_LEAKY = 0.1
_EPS = 1e-5


def _lk(x):
    return jnp.where(x >= 0, x, _LEAKY * x)


_FPS_R, _FPS_C = 64, 128


def _fps_body(x_ref, idx_ref, nxyz_ref):
    x = x_ref[0, 0]
    y = x_ref[0, 1]
    z = x_ref[0, 2]
    flat = (jax.lax.broadcasted_iota(jnp.int32, (_FPS_R, _FPS_C), 0) * _FPS_C
            + jax.lax.broadcasted_iota(jnp.int32, (_FPS_R, _FPS_C), 1))

    def body(i, carry):
        dists, far = carry
        idx_ref[0, 0, i] = far
        mask = flat == far
        cx = jnp.sum(jnp.where(mask, x, 0.0))
        cy = jnp.sum(jnp.where(mask, y, 0.0))
        cz = jnp.sum(jnp.where(mask, z, 0.0))
        nxyz_ref[0, 0, i] = cx
        nxyz_ref[0, 1, i] = cy
        nxyz_ref[0, 2, i] = cz
        dx = x - cx
        dy = y - cy
        dz = z - cz
        d = dx * dx + dy * dy + dz * dz
        dists = jnp.minimum(dists, d)
        m = jnp.max(dists)
        far = jnp.min(jnp.where(dists == m, flat, jnp.int32(_N)))
        return (dists, far)

    dists0 = jnp.full((_FPS_R, _FPS_C), 1e10, jnp.float32)
    jax.lax.fori_loop(0, _S, body, (dists0, jnp.int32(0)))


def _fps_pallas(xyz):
    xr = xyz.reshape(_B, 3, _FPS_R, _FPS_C)
    idx, nxyz = pl.pallas_call(
        _fps_body,
        grid=(_B,),
        in_specs=[pl.BlockSpec((1, 3, _FPS_R, _FPS_C), lambda b: (b, 0, 0, 0))],
        out_specs=[
            pl.BlockSpec((1, 1, _S), lambda b: (b, 0, 0), memory_space=pltpu.SMEM),
            pl.BlockSpec((1, 3, _S), lambda b: (b, 0, 0), memory_space=pltpu.SMEM),
        ],
        out_shape=[
            jax.ShapeDtypeStruct((_B, 1, _S), jnp.int32),
            jax.ShapeDtypeStruct((_B, 3, _S), jnp.float32),
        ],
    )(xr)
    return idx.reshape(_B, _S), nxyz


_KNN_ROWS = 32
_NCHUNK = _N // 128


def _batcher_pairs(n):
    pairs = []
    p = 1
    while p < n:
        k = p
        while k >= 1:
            for j in range(k % p, n - k, 2 * k):
                for i in range(0, min(k, n - j - k)):
                    if (i + j) // (2 * p) == (i + j + k) // (2 * p):
                        pairs.append((i + j, i + j + k))
            k //= 2
        p *= 2
    return pairs


def _pruned_net(n, keep):
    needed = set(range(keep))
    kept = []
    for (i, j) in reversed(_batcher_pairs(n)):
        if i in needed or j in needed:
            kept.append((i, j))
            needed.add(i)
            needed.add(j)
    kept.reverse()
    return kept


_KNN_NET = _pruned_net(_NCHUNK, _K)


def _knn_body(c_ref, x_ref, idx_ref):
    rows = _KNN_ROWS
    cb = c_ref[0]
    xb = x_ref[0]
    mm = jnp.dot(cb, xb, preferred_element_type=jnp.float32)
    cn = jnp.sum(cb * cb, axis=1, keepdims=True)
    xn = jnp.sum(xb * xb, axis=0, keepdims=True)
    d = -2.0 * mm
    d = d + cn
    d = d + xn

    keys = [d[:, 128 * c:128 * (c + 1)] for c in range(_NCHUNK)]
    lane = jax.lax.broadcasted_iota(jnp.int32, (rows, 128), 1)
    pay = [lane + 128 * c for c in range(_NCHUNK)]
    for (i, j) in _KNN_NET:
        a, b = keys[i], keys[j]
        m = a <= b
        keys[i] = jnp.minimum(a, b)
        keys[j] = jnp.maximum(a, b)
        pi, pj = pay[i], pay[j]
        pay[i] = jnp.where(m, pi, pj)
        pay[j] = jnp.where(m, pj, pi)

    F = keys[0]
    FI = pay[0]
    ptr = jnp.zeros((rows, 128), jnp.int32)
    lane_k = jax.lax.broadcasted_iota(jnp.int32, (rows, _K), 1)
    out = jnp.zeros((rows, _K), jnp.int32)
    for kk in range(_K):
        m = jnp.min(F, axis=1, keepdims=True)
        sel = F == m
        lsel = jnp.min(jnp.where(sel, lane, _N), axis=1, keepdims=True)
        lmask = lane == lsel
        ei = jnp.min(jnp.where(lmask, FI, _N), axis=1, keepdims=True)
        out = jnp.where(lane_k == kk, ei, out)
        if kk < _K - 1:
            ptr = ptr + lmask.astype(jnp.int32)
            depth = kk + 1
            nk = keys[depth]
            ni = pay[depth]
            for r in range(depth - 1, 0, -1):
                selr = ptr == r
                nk = jnp.where(selr, keys[r], nk)
                ni = jnp.where(selr, pay[r], ni)
            F = jnp.where(lmask, nk, F)
            FI = jnp.where(lmask, ni, FI)
    idx_ref[0] = out


def _knn_pallas(new_xyz, xyz):
    return pl.pallas_call(
        _knn_body,
        grid=(_B, _S // _KNN_ROWS),
        in_specs=[
            pl.BlockSpec((1, _KNN_ROWS, 3), lambda b, s: (b, s, 0)),
            pl.BlockSpec((1, 3, _N), lambda b, s: (b, 0, 0)),
        ],
        out_specs=pl.BlockSpec((1, _KNN_ROWS, _K), lambda b, s: (b, s, 0)),
        out_shape=jax.ShapeDtypeStruct((_B, _S, _K), jnp.int32),
    )(new_xyz, xyz)


_TROWS = 256

_SC_NW = 32
_SC_CHUNK = 128
_SC_PER_W = (_B * _S * _K) // _SC_NW


def _sc_gather(table, idxs):
    mesh = plsc.VectorSubcoreMesh(core_axis_name="c", subcore_axis_name="s")

    @functools.partial(
        pl.kernel, mesh=mesh,
        out_type=jax.ShapeDtypeStruct((_B * _S * _K, 128), jnp.float32),
        scratch_types=[
            pltpu.VMEM((_SC_PER_W,), jnp.int32),
            pltpu.VMEM((_SC_CHUNK, 128), jnp.float32),
            pltpu.SemaphoreType.DMA,
        ],
    )
    def k(table_hbm, idx_hbm, out_hbm, idx_v, rows_v, sem):
        wid = jax.lax.axis_index("s") * 2 + jax.lax.axis_index("c")
        base = wid * _SC_PER_W
        pltpu.sync_copy(idx_hbm.at[pl.ds(base, _SC_PER_W)], idx_v)

        def body(j, carry):
            off = j * _SC_CHUNK
            pltpu.async_copy(
                table_hbm.at[idx_v.at[pl.ds(off, _SC_CHUNK)]], rows_v, sem
            ).wait()
            pltpu.sync_copy(rows_v, out_hbm.at[pl.ds(base + off, _SC_CHUNK)])
            return carry

        jax.lax.fori_loop(0, _SC_PER_W // _SC_CHUNK, body, 0)

    return k(table, idxs)


def _gmat_kernel(in_ref, w_ref, out_ref):
    out_ref[0] = jnp.dot(in_ref[0], w_ref[...],
                         preferred_element_type=jnp.float32, precision=jax.lax.Precision.HIGHEST)


def _stats_kernel(g_ref, c_ref, wm_ref, out_ref):
    step = pl.program_id(0)
    c64 = jnp.dot(c_ref[...], wm_ref[...], preferred_element_type=jnp.float32, precision=jax.lax.Precision.HIGHEST)
    x = g_ref[...][:, :, :_COUT] + c64[:, None, :]
    s1 = jnp.sum(x, axis=(0, 1))[None, :]
    s2 = jnp.sum(x * x, axis=(0, 1))[None, :]
    acc = jnp.concatenate([s1, s2], axis=0)

    @pl.when(step == 0)
    def _():
        out_ref[...] = jnp.zeros_like(out_ref)

    out_ref[...] += acc


def _main_kernel(g_ref, c_ref, wm_ref, sc_ref, sh_ref, wst_ref, w1t_ref,
                 f1_ref, st_ref):
    step = pl.program_id(0)
    c64 = jnp.dot(c_ref[...], wm_ref[...], preferred_element_type=jnp.float32, precision=jax.lax.Precision.HIGHEST)
    x = g_ref[...][:, :, :_COUT] + c64[:, None, :]
    lse1 = x * sc_ref[0][None, None, :] + sh_ref[0][None, None, :]
    lse1 = jnp.where(lse1 >= 0, lse1, _LEAKY * lse1)
    l2 = lse1.reshape(_TROWS * _K, _COUT)
    z = jnp.dot(l2, wst_ref[...], preferred_element_type=jnp.float32, precision=jax.lax.Precision.HIGHEST)
    z = jnp.where(z >= 0, z, _LEAKY * z).reshape(_TROWS, _K, _COUT)
    zm = jnp.max(z, axis=1, keepdims=True)
    e = jnp.exp(z - zm)
    sc = e / jnp.sum(e, axis=1, keepdims=True)
    feat = jnp.sum(sc * lse1, axis=1)
    f1 = jnp.dot(feat, w1t_ref[...], preferred_element_type=jnp.float32, precision=jax.lax.Precision.HIGHEST)
    f1_ref[...] = f1
    s1 = jnp.sum(f1, axis=0)[None, :]
    s2 = jnp.sum(f1 * f1, axis=0)[None, :]
    acc = jnp.concatenate([s1, s2], axis=0)

    @pl.when(step == 0)
    def _():
        st_ref[...] = jnp.zeros_like(st_ref)

    st_ref[...] += acc


def _final_kernel(f1_ref, sc_ref, sh_ref, out_ref):
    y = f1_ref[...] * sc_ref[0][None, :] + sh_ref[0][None, :]
    out_ref[...] = jnp.where(y >= 0, y, _LEAKY * y)


def kernel(xyz, points, W0, b0, g0, be0, Wl, bl, gl, bel, Ws, W1, b1, g1, be1):
    xyz_t = xyz.transpose(0, 2, 1)
    pts_t = points.transpose(0, 2, 1)
    fps_idx, nxyz_cs = _fps_pallas(xyz)
    new_xyz = nxyz_cs.transpose(0, 2, 1)

    idx = _knn_pallas(new_xyz, xyz)

    half = _COUT // 2
    Wg = Wl[:, 3:6] + Wl[:, 6:9]
    Wm = Wl[:, 0:3] - Wl[:, 6:9]
    Wcomb = jnp.zeros((3 + _CIN, 128), jnp.float32)
    Wcomb = Wcomb.at[0:3, 0:half].set(Wg.T)
    Wcomb = Wcomb.at[3:, half:_COUT].set(W0.T)
    in2 = jnp.concatenate([xyz_t, pts_t], axis=-1)
    G = pl.pallas_call(
        _gmat_kernel,
        grid=(_B,),
        in_specs=[pl.BlockSpec((1, _N, 3 + _CIN), lambda b: (b, 0, 0)),
                  pl.BlockSpec((3 + _CIN, 128), lambda b: (0, 0))],
        out_specs=pl.BlockSpec((1, _N, 128), lambda b: (b, 0, 0)),
        out_shape=jax.ShapeDtypeStruct((_B, _N, 128), jnp.float32),
    )(in2, Wcomb)

    idx_glob = (idx + (jnp.arange(_B, dtype=jnp.int32) * _N)[:, None, None])
    Gg = _sc_gather(G.reshape(_B * _N, 128), idx_glob.reshape(-1))
    Gg = Gg.reshape(_B * _S, _K, 128)

    nxyz_flat = new_xyz.reshape(_B * _S, 3)
    Wm64 = jnp.zeros((3, _COUT), jnp.float32).at[:, 0:half].set(Wm.T)

    nsteps = (_B * _S) // _TROWS
    row_spec = pl.BlockSpec((_TROWS, _K, 128), lambda s: (s, 0, 0))
    c_spec = pl.BlockSpec((_TROWS, 3), lambda s: (s, 0))
    wm_spec = pl.BlockSpec((3, _COUT), lambda s: (0, 0))
    acc_spec = pl.BlockSpec((2, _COUT), lambda s: (0, 0))
    vec_spec = pl.BlockSpec((1, _COUT), lambda s: (0, 0))
    w64_spec = pl.BlockSpec((_COUT, _COUT), lambda s: (0, 0))

    sums = pl.pallas_call(
        _stats_kernel,
        grid=(nsteps,),
        in_specs=[row_spec, c_spec, wm_spec],
        out_specs=acc_spec,
        out_shape=jax.ShapeDtypeStruct((2, _COUT), jnp.float32),
    )(Gg, nxyz_flat, Wm64)

    n = float(_B * _S * _K)
    gf = jnp.concatenate([gl, g0])
    bef = jnp.concatenate([bel, be0])
    mean_x = sums[0] / n
    var = sums[1] / n - mean_x * mean_x
    scale64 = gf / jnp.sqrt(var + _EPS)
    shift64 = bef - mean_x * scale64

    f1, st = pl.pallas_call(
        _main_kernel,
        grid=(nsteps,),
        in_specs=[row_spec, c_spec, wm_spec, vec_spec, vec_spec,
                  w64_spec, w64_spec],
        out_specs=[pl.BlockSpec((_TROWS, _COUT), lambda s: (s, 0)), acc_spec],
        out_shape=[jax.ShapeDtypeStruct((_B * _S, _COUT), jnp.float32),
                   jax.ShapeDtypeStruct((2, _COUT), jnp.float32)],
    )(Gg, nxyz_flat, Wm64, scale64[None], shift64[None], Ws.T, W1.T)

    n1 = float(_B * _S)
    mean1 = st[0] / n1
    var1 = st[1] / n1 - mean1 * mean1
    scale1 = g1 / jnp.sqrt(var1 + _EPS)
    shift1 = be1 - mean1 * scale1

    outp = pl.pallas_call(
        _final_kernel,
        grid=(_B,),
        in_specs=[pl.BlockSpec((_S, _COUT), lambda b: (b, 0)),
                  pl.BlockSpec((1, _COUT), lambda b: (0, 0)),
                  pl.BlockSpec((1, _COUT), lambda b: (0, 0))],
        out_specs=pl.BlockSpec((_S, _COUT), lambda b: (b, 0)),
        out_shape=jax.ShapeDtypeStruct((_B * _S, _COUT), jnp.float32),
    )(f1, scale1[None], shift1[None])

    out_points = outp.reshape(_B, _S, _COUT).transpose(0, 2, 1)
    return (nxyz_cs, out_points, fps_idx)


# speedup vs baseline: 3.4841x; 1.1615x over previous
import functools

import jax
import jax.numpy as jnp
from jax.experimental import pallas as pl
from jax.experimental.pallas import tpu as pltpu
from jax.experimental.pallas import tpu_sc as plsc

_B, _N, _S, _K = 2, 8192, 2048, 32
_CIN, _COUT = 64, 64
_LEAKY = 0.1
_EPS = 1e-5


def _lk(x):
    return jnp.where(x >= 0, x, _LEAKY * x)


_FPS_R, _FPS_C = 64, 128


def _fps_body(x_ref, idx_ref, nxyz_ref):
    x = x_ref[0, 0]
    y = x_ref[0, 1]
    z = x_ref[0, 2]
    flat = (jax.lax.broadcasted_iota(jnp.int32, (_FPS_R, _FPS_C), 0) * _FPS_C
            + jax.lax.broadcasted_iota(jnp.int32, (_FPS_R, _FPS_C), 1))

    def body(i, carry):
        dists, far = carry
        idx_ref[0, 0, i] = far
        mask = flat == far
        cx = jnp.sum(jnp.where(mask, x, 0.0))
        cy = jnp.sum(jnp.where(mask, y, 0.0))
        cz = jnp.sum(jnp.where(mask, z, 0.0))
        nxyz_ref[0, 0, i] = cx
        nxyz_ref[0, 1, i] = cy
        nxyz_ref[0, 2, i] = cz
        dx = x - cx
        dy = y - cy
        dz = z - cz
        d = dx * dx + dy * dy + dz * dz
        dists = jnp.minimum(dists, d)
        m = jnp.max(dists)
        far = jnp.min(jnp.where(dists == m, flat, jnp.int32(_N)))
        return (dists, far)

    dists0 = jnp.full((_FPS_R, _FPS_C), 1e10, jnp.float32)
    jax.lax.fori_loop(0, _S, body, (dists0, jnp.int32(0)))


def _fps_pallas(xyz):
    xr = xyz.reshape(_B, 3, _FPS_R, _FPS_C)
    idx, nxyz = pl.pallas_call(
        _fps_body,
        grid=(_B,),
        in_specs=[pl.BlockSpec((1, 3, _FPS_R, _FPS_C), lambda b: (b, 0, 0, 0))],
        out_specs=[
            pl.BlockSpec((1, 1, _S), lambda b: (b, 0, 0), memory_space=pltpu.SMEM),
            pl.BlockSpec((1, 3, _S), lambda b: (b, 0, 0), memory_space=pltpu.SMEM),
        ],
        out_shape=[
            jax.ShapeDtypeStruct((_B, 1, _S), jnp.int32),
            jax.ShapeDtypeStruct((_B, 3, _S), jnp.float32),
        ],
    )(xr)
    return idx.reshape(_B, _S), nxyz


_KNN_ROWS = 64
_NCHUNK = _N // 128


def _batcher_pairs(n):
    pairs = []
    p = 1
    while p < n:
        k = p
        while k >= 1:
            for j in range(k % p, n - k, 2 * k):
                for i in range(0, min(k, n - j - k)):
                    if (i + j) // (2 * p) == (i + j + k) // (2 * p):
                        pairs.append((i + j, i + j + k))
            k //= 2
        p *= 2
    return pairs


def _pruned_net(n, keep):
    needed = set(range(keep))
    kept = []
    for (i, j) in reversed(_batcher_pairs(n)):
        if i in needed or j in needed:
            kept.append((i, j))
            needed.add(i)
            needed.add(j)
    kept.reverse()
    return kept


_KNN_NET = _pruned_net(_NCHUNK, _K)


def _knn_body(c_ref, x_ref, idx_ref):
    rows = _KNN_ROWS
    cb = c_ref[0]
    xb = x_ref[0]
    mm = jnp.dot(cb, xb, preferred_element_type=jnp.float32)
    cn = jnp.sum(cb * cb, axis=1, keepdims=True)
    xn = jnp.sum(xb * xb, axis=0, keepdims=True)
    d = -2.0 * mm
    d = d + cn
    d = d + xn

    keys = [d[:, 128 * c:128 * (c + 1)] for c in range(_NCHUNK)]
    lane = jax.lax.broadcasted_iota(jnp.int32, (rows, 128), 1)
    pay = [lane + 128 * c for c in range(_NCHUNK)]
    for (i, j) in _KNN_NET:
        a, b = keys[i], keys[j]
        m = a <= b
        keys[i] = jnp.minimum(a, b)
        keys[j] = jnp.maximum(a, b)
        pi, pj = pay[i], pay[j]
        pay[i] = jnp.where(m, pi, pj)
        pay[j] = jnp.where(m, pj, pi)

    F = keys[0]
    FI = pay[0]
    ptr = jnp.zeros((rows, 128), jnp.int32)
    lane_k = jax.lax.broadcasted_iota(jnp.int32, (rows, _K), 1)
    out = jnp.zeros((rows, _K), jnp.int32)
    for kk in range(_K):
        m = jnp.min(F, axis=1, keepdims=True)
        sel = F == m
        lsel = jnp.min(jnp.where(sel, lane, _N), axis=1, keepdims=True)
        lmask = lane == lsel
        ei = jnp.min(jnp.where(lmask, FI, _N), axis=1, keepdims=True)
        out = jnp.where(lane_k == kk, ei, out)
        if kk < _K - 1:
            ptr = ptr + lmask.astype(jnp.int32)
            depth = kk + 1
            nk = keys[depth]
            ni = pay[depth]
            for r in range(depth - 1, 0, -1):
                selr = ptr == r
                nk = jnp.where(selr, keys[r], nk)
                ni = jnp.where(selr, pay[r], ni)
            F = jnp.where(lmask, nk, F)
            FI = jnp.where(lmask, ni, FI)
    idx_ref[0] = out


def _knn_pallas(new_xyz, xyz):
    return pl.pallas_call(
        _knn_body,
        grid=(_B, _S // _KNN_ROWS),
        in_specs=[
            pl.BlockSpec((1, _KNN_ROWS, 3), lambda b, s: (b, s, 0)),
            pl.BlockSpec((1, 3, _N), lambda b, s: (b, 0, 0)),
        ],
        out_specs=pl.BlockSpec((1, _KNN_ROWS, _K), lambda b, s: (b, s, 0)),
        out_shape=jax.ShapeDtypeStruct((_B, _S, _K), jnp.int32),
    )(new_xyz, xyz)


_TROWS = 256

_SC_NW = 32
_SC_CHUNK = 128
_SC_PER_W = (_B * _S * _K) // _SC_NW


def _sc_gather(table, idxs):
    mesh = plsc.VectorSubcoreMesh(core_axis_name="c", subcore_axis_name="s")

    @functools.partial(
        pl.kernel, mesh=mesh,
        out_type=jax.ShapeDtypeStruct((_B * _S * _K, 128), jnp.float32),
        scratch_types=[
            pltpu.VMEM((_SC_PER_W,), jnp.int32),
            pltpu.VMEM((_SC_CHUNK, 128), jnp.float32),
            pltpu.SemaphoreType.DMA,
        ],
    )
    def k(table_hbm, idx_hbm, out_hbm, idx_v, rows_v, sem):
        wid = jax.lax.axis_index("s") * 2 + jax.lax.axis_index("c")
        base = wid * _SC_PER_W
        pltpu.sync_copy(idx_hbm.at[pl.ds(base, _SC_PER_W)], idx_v)

        def body(j, carry):
            off = j * _SC_CHUNK
            pltpu.async_copy(
                table_hbm.at[idx_v.at[pl.ds(off, _SC_CHUNK)]], rows_v, sem
            ).wait()
            pltpu.sync_copy(rows_v, out_hbm.at[pl.ds(base + off, _SC_CHUNK)])
            return carry

        jax.lax.fori_loop(0, _SC_PER_W // _SC_CHUNK, body, 0)

    return k(table, idxs)


def _gmat_kernel(in_ref, w_ref, out_ref):
    out_ref[0] = jnp.dot(in_ref[0], w_ref[...],
                         preferred_element_type=jnp.float32, precision=jax.lax.Precision.HIGHEST)


def _stats_kernel(g_ref, c_ref, wm_ref, out_ref):
    step = pl.program_id(0)
    c64 = jnp.dot(c_ref[...], wm_ref[...], preferred_element_type=jnp.float32, precision=jax.lax.Precision.HIGHEST)
    x = g_ref[...][:, :, :_COUT] + c64[:, None, :]
    s1 = jnp.sum(x, axis=(0, 1))[None, :]
    s2 = jnp.sum(x * x, axis=(0, 1))[None, :]
    acc = jnp.concatenate([s1, s2], axis=0)

    @pl.when(step == 0)
    def _():
        out_ref[...] = jnp.zeros_like(out_ref)

    out_ref[...] += acc


def _main_kernel(g_ref, c_ref, wm_ref, sc_ref, sh_ref, wst_ref, w1t_ref,
                 f1_ref, st_ref):
    step = pl.program_id(0)
    c64 = jnp.dot(c_ref[...], wm_ref[...], preferred_element_type=jnp.float32, precision=jax.lax.Precision.HIGHEST)
    x = g_ref[...][:, :, :_COUT] + c64[:, None, :]
    lse1 = x * sc_ref[0][None, None, :] + sh_ref[0][None, None, :]
    lse1 = jnp.where(lse1 >= 0, lse1, _LEAKY * lse1)
    l2 = lse1.reshape(_TROWS * _K, _COUT)
    z = jnp.dot(l2, wst_ref[...], preferred_element_type=jnp.float32, precision=jax.lax.Precision.HIGHEST)
    z = jnp.where(z >= 0, z, _LEAKY * z).reshape(_TROWS, _K, _COUT)
    zm = jnp.max(z, axis=1, keepdims=True)
    e = jnp.exp(z - zm)
    sc = e / jnp.sum(e, axis=1, keepdims=True)
    feat = jnp.sum(sc * lse1, axis=1)
    f1 = jnp.dot(feat, w1t_ref[...], preferred_element_type=jnp.float32, precision=jax.lax.Precision.HIGHEST)
    f1_ref[...] = f1
    s1 = jnp.sum(f1, axis=0)[None, :]
    s2 = jnp.sum(f1 * f1, axis=0)[None, :]
    acc = jnp.concatenate([s1, s2], axis=0)

    @pl.when(step == 0)
    def _():
        st_ref[...] = jnp.zeros_like(st_ref)

    st_ref[...] += acc


def _final_kernel(f1_ref, sc_ref, sh_ref, out_ref):
    y = f1_ref[...] * sc_ref[0][None, :] + sh_ref[0][None, :]
    out_ref[...] = jnp.where(y >= 0, y, _LEAKY * y)


def kernel(xyz, points, W0, b0, g0, be0, Wl, bl, gl, bel, Ws, W1, b1, g1, be1):
    xyz_t = xyz.transpose(0, 2, 1)
    pts_t = points.transpose(0, 2, 1)
    fps_idx, nxyz_cs = _fps_pallas(xyz)
    new_xyz = nxyz_cs.transpose(0, 2, 1)

    idx = _knn_pallas(new_xyz, xyz)

    half = _COUT // 2
    Wg = Wl[:, 3:6] + Wl[:, 6:9]
    Wm = Wl[:, 0:3] - Wl[:, 6:9]
    Wcomb = jnp.zeros((3 + _CIN, 128), jnp.float32)
    Wcomb = Wcomb.at[0:3, 0:half].set(Wg.T)
    Wcomb = Wcomb.at[3:, half:_COUT].set(W0.T)
    in2 = jnp.concatenate([xyz_t, pts_t], axis=-1)
    G = pl.pallas_call(
        _gmat_kernel,
        grid=(_B,),
        in_specs=[pl.BlockSpec((1, _N, 3 + _CIN), lambda b: (b, 0, 0)),
                  pl.BlockSpec((3 + _CIN, 128), lambda b: (0, 0))],
        out_specs=pl.BlockSpec((1, _N, 128), lambda b: (b, 0, 0)),
        out_shape=jax.ShapeDtypeStruct((_B, _N, 128), jnp.float32),
    )(in2, Wcomb)

    idx_glob = (idx + (jnp.arange(_B, dtype=jnp.int32) * _N)[:, None, None])
    Gg = _sc_gather(G.reshape(_B * _N, 128), idx_glob.reshape(-1))
    Gg = Gg.reshape(_B * _S, _K, 128)

    nxyz_flat = new_xyz.reshape(_B * _S, 3)
    Wm64 = jnp.zeros((3, _COUT), jnp.float32).at[:, 0:half].set(Wm.T)

    nsteps = (_B * _S) // _TROWS
    row_spec = pl.BlockSpec((_TROWS, _K, 128), lambda s: (s, 0, 0))
    c_spec = pl.BlockSpec((_TROWS, 3), lambda s: (s, 0))
    wm_spec = pl.BlockSpec((3, _COUT), lambda s: (0, 0))
    acc_spec = pl.BlockSpec((2, _COUT), lambda s: (0, 0))
    vec_spec = pl.BlockSpec((1, _COUT), lambda s: (0, 0))
    w64_spec = pl.BlockSpec((_COUT, _COUT), lambda s: (0, 0))

    sums = pl.pallas_call(
        _stats_kernel,
        grid=(nsteps,),
        in_specs=[row_spec, c_spec, wm_spec],
        out_specs=acc_spec,
        out_shape=jax.ShapeDtypeStruct((2, _COUT), jnp.float32),
    )(Gg, nxyz_flat, Wm64)

    n = float(_B * _S * _K)
    gf = jnp.concatenate([gl, g0])
    bef = jnp.concatenate([bel, be0])
    mean_x = sums[0] / n
    var = sums[1] / n - mean_x * mean_x
    scale64 = gf / jnp.sqrt(var + _EPS)
    shift64 = bef - mean_x * scale64

    f1, st = pl.pallas_call(
        _main_kernel,
        grid=(nsteps,),
        in_specs=[row_spec, c_spec, wm_spec, vec_spec, vec_spec,
                  w64_spec, w64_spec],
        out_specs=[pl.BlockSpec((_TROWS, _COUT), lambda s: (s, 0)), acc_spec],
        out_shape=[jax.ShapeDtypeStruct((_B * _S, _COUT), jnp.float32),
                   jax.ShapeDtypeStruct((2, _COUT), jnp.float32)],
    )(Gg, nxyz_flat, Wm64, scale64[None], shift64[None], Ws.T, W1.T)

    n1 = float(_B * _S)
    mean1 = st[0] / n1
    var1 = st[1] / n1 - mean1 * mean1
    scale1 = g1 / jnp.sqrt(var1 + _EPS)
    shift1 = be1 - mean1 * scale1

    outp = pl.pallas_call(
        _final_kernel,
        grid=(_B,),
        in_specs=[pl.BlockSpec((_S, _COUT), lambda b: (b, 0)),
                  pl.BlockSpec((1, _COUT), lambda b: (0, 0)),
                  pl.BlockSpec((1, _COUT), lambda b: (0, 0))],
        out_specs=pl.BlockSpec((_S, _COUT), lambda b: (b, 0)),
        out_shape=jax.ShapeDtypeStruct((_B * _S, _COUT), jnp.float32),
    )(f1, scale1[None], shift1[None])

    out_points = outp.reshape(_B, _S, _COUT).transpose(0, 2, 1)
    return (nxyz_cs, out_points, fps_idx)


# speedup vs baseline: 3.7552x; 1.0778x over previous
import functools

import jax
import jax.numpy as jnp
from jax.experimental import pallas as pl
from jax.experimental.pallas import tpu as pltpu
from jax.experimental.pallas import tpu_sc as plsc

_B, _N, _S, _K = 2, 8192, 2048, 32
_CIN, _COUT = 64, 64
_LEAKY = 0.1
_EPS = 1e-5


def _lk(x):
    return jnp.where(x >= 0, x, _LEAKY * x)


_FPS_R, _FPS_C = 64, 128


def _fps_body(x_ref, idx_ref, nxyz_ref):
    x = x_ref[0, 0]
    y = x_ref[0, 1]
    z = x_ref[0, 2]
    flat = (jax.lax.broadcasted_iota(jnp.int32, (_FPS_R, _FPS_C), 0) * _FPS_C
            + jax.lax.broadcasted_iota(jnp.int32, (_FPS_R, _FPS_C), 1))

    def body(i, carry):
        dists, far = carry
        idx_ref[0, 0, i] = far
        mask = flat == far
        cx = jnp.sum(jnp.where(mask, x, 0.0))
        cy = jnp.sum(jnp.where(mask, y, 0.0))
        cz = jnp.sum(jnp.where(mask, z, 0.0))
        nxyz_ref[0, 0, i] = cx
        nxyz_ref[0, 1, i] = cy
        nxyz_ref[0, 2, i] = cz
        dx = x - cx
        dy = y - cy
        dz = z - cz
        d = dx * dx + dy * dy + dz * dz
        dists = jnp.minimum(dists, d)
        m = jnp.max(dists)
        far = jnp.min(jnp.where(dists == m, flat, jnp.int32(_N)))
        return (dists, far)

    dists0 = jnp.full((_FPS_R, _FPS_C), 1e10, jnp.float32)
    jax.lax.fori_loop(0, _S, body, (dists0, jnp.int32(0)))


def _fps_pallas(xyz):
    xr = xyz.reshape(_B, 3, _FPS_R, _FPS_C)
    idx, nxyz = pl.pallas_call(
        _fps_body,
        grid=(_B,),
        in_specs=[pl.BlockSpec((1, 3, _FPS_R, _FPS_C), lambda b: (b, 0, 0, 0))],
        out_specs=[
            pl.BlockSpec((1, 1, _S), lambda b: (b, 0, 0), memory_space=pltpu.SMEM),
            pl.BlockSpec((1, 3, _S), lambda b: (b, 0, 0), memory_space=pltpu.SMEM),
        ],
        out_shape=[
            jax.ShapeDtypeStruct((_B, 1, _S), jnp.int32),
            jax.ShapeDtypeStruct((_B, 3, _S), jnp.float32),
        ],
    )(xr)
    return idx.reshape(_B, _S), nxyz


_KNN_ROWS = 128
_NCHUNK = _N // 128


def _batcher_pairs(n):
    pairs = []
    p = 1
    while p < n:
        k = p
        while k >= 1:
            for j in range(k % p, n - k, 2 * k):
                for i in range(0, min(k, n - j - k)):
                    if (i + j) // (2 * p) == (i + j + k) // (2 * p):
                        pairs.append((i + j, i + j + k))
            k //= 2
        p *= 2
    return pairs


def _pruned_net(n, keep):
    needed = set(range(keep))
    kept = []
    for (i, j) in reversed(_batcher_pairs(n)):
        if i in needed or j in needed:
            kept.append((i, j))
            needed.add(i)
            needed.add(j)
    kept.reverse()
    return kept


_KNN_NET = _pruned_net(_NCHUNK, _K)


def _knn_body(c_ref, x_ref, idx_ref):
    rows = _KNN_ROWS
    cb = c_ref[0]
    xb = x_ref[0]
    mm = jnp.dot(cb, xb, preferred_element_type=jnp.float32)
    cn = jnp.sum(cb * cb, axis=1, keepdims=True)
    xn = jnp.sum(xb * xb, axis=0, keepdims=True)
    d = -2.0 * mm
    d = d + cn
    d = d + xn

    keys = [d[:, 128 * c:128 * (c + 1)] for c in range(_NCHUNK)]
    lane = jax.lax.broadcasted_iota(jnp.int32, (rows, 128), 1)
    pay = [lane + 128 * c for c in range(_NCHUNK)]
    for (i, j) in _KNN_NET:
        a, b = keys[i], keys[j]
        m = a <= b
        keys[i] = jnp.minimum(a, b)
        keys[j] = jnp.maximum(a, b)
        pi, pj = pay[i], pay[j]
        pay[i] = jnp.where(m, pi, pj)
        pay[j] = jnp.where(m, pj, pi)

    F = keys[0]
    FI = pay[0]
    ptr = jnp.zeros((rows, 128), jnp.int32)
    lane_k = jax.lax.broadcasted_iota(jnp.int32, (rows, _K), 1)
    out = jnp.zeros((rows, _K), jnp.int32)
    for kk in range(_K):
        m = jnp.min(F, axis=1, keepdims=True)
        sel = F == m
        lsel = jnp.min(jnp.where(sel, lane, _N), axis=1, keepdims=True)
        lmask = lane == lsel
        ei = jnp.min(jnp.where(lmask, FI, _N), axis=1, keepdims=True)
        out = jnp.where(lane_k == kk, ei, out)
        if kk < _K - 1:
            ptr = ptr + lmask.astype(jnp.int32)
            depth = kk + 1
            nk = keys[depth]
            ni = pay[depth]
            for r in range(depth - 1, 0, -1):
                selr = ptr == r
                nk = jnp.where(selr, keys[r], nk)
                ni = jnp.where(selr, pay[r], ni)
            F = jnp.where(lmask, nk, F)
            FI = jnp.where(lmask, ni, FI)
    idx_ref[0] = out


def _knn_pallas(new_xyz, xyz):
    return pl.pallas_call(
        _knn_body,
        grid=(_B, _S // _KNN_ROWS),
        in_specs=[
            pl.BlockSpec((1, _KNN_ROWS, 3), lambda b, s: (b, s, 0)),
            pl.BlockSpec((1, 3, _N), lambda b, s: (b, 0, 0)),
        ],
        out_specs=pl.BlockSpec((1, _KNN_ROWS, _K), lambda b, s: (b, s, 0)),
        out_shape=jax.ShapeDtypeStruct((_B, _S, _K), jnp.int32),
    )(new_xyz, xyz)


_TROWS = 256

_SC_NW = 32
_SC_CHUNK = 128
_SC_PER_W = (_B * _S * _K) // _SC_NW


def _sc_gather(table, idxs):
    mesh = plsc.VectorSubcoreMesh(core_axis_name="c", subcore_axis_name="s")

    @functools.partial(
        pl.kernel, mesh=mesh,
        out_type=jax.ShapeDtypeStruct((_B * _S * _K, 128), jnp.float32),
        scratch_types=[
            pltpu.VMEM((_SC_PER_W,), jnp.int32),
            pltpu.VMEM((_SC_CHUNK, 128), jnp.float32),
            pltpu.SemaphoreType.DMA,
        ],
    )
    def k(table_hbm, idx_hbm, out_hbm, idx_v, rows_v, sem):
        wid = jax.lax.axis_index("s") * 2 + jax.lax.axis_index("c")
        base = wid * _SC_PER_W
        pltpu.sync_copy(idx_hbm.at[pl.ds(base, _SC_PER_W)], idx_v)

        def body(j, carry):
            off = j * _SC_CHUNK
            pltpu.async_copy(
                table_hbm.at[idx_v.at[pl.ds(off, _SC_CHUNK)]], rows_v, sem
            ).wait()
            pltpu.sync_copy(rows_v, out_hbm.at[pl.ds(base + off, _SC_CHUNK)])
            return carry

        jax.lax.fori_loop(0, _SC_PER_W // _SC_CHUNK, body, 0)

    return k(table, idxs)


def _gmat_kernel(in_ref, w_ref, out_ref):
    out_ref[0] = jnp.dot(in_ref[0], w_ref[...],
                         preferred_element_type=jnp.float32, precision=jax.lax.Precision.HIGHEST)


def _stats_kernel(g_ref, c_ref, wm_ref, out_ref):
    step = pl.program_id(0)
    c64 = jnp.dot(c_ref[...], wm_ref[...], preferred_element_type=jnp.float32, precision=jax.lax.Precision.HIGHEST)
    x = g_ref[...][:, :, :_COUT] + c64[:, None, :]
    s1 = jnp.sum(x, axis=(0, 1))[None, :]
    s2 = jnp.sum(x * x, axis=(0, 1))[None, :]
    acc = jnp.concatenate([s1, s2], axis=0)

    @pl.when(step == 0)
    def _():
        out_ref[...] = jnp.zeros_like(out_ref)

    out_ref[...] += acc


def _main_kernel(g_ref, c_ref, wm_ref, sc_ref, sh_ref, wst_ref, w1t_ref,
                 f1_ref, st_ref):
    step = pl.program_id(0)
    c64 = jnp.dot(c_ref[...], wm_ref[...], preferred_element_type=jnp.float32, precision=jax.lax.Precision.HIGHEST)
    x = g_ref[...][:, :, :_COUT] + c64[:, None, :]
    lse1 = x * sc_ref[0][None, None, :] + sh_ref[0][None, None, :]
    lse1 = jnp.where(lse1 >= 0, lse1, _LEAKY * lse1)
    l2 = lse1.reshape(_TROWS * _K, _COUT)
    z = jnp.dot(l2, wst_ref[...], preferred_element_type=jnp.float32, precision=jax.lax.Precision.HIGHEST)
    z = jnp.where(z >= 0, z, _LEAKY * z).reshape(_TROWS, _K, _COUT)
    zm = jnp.max(z, axis=1, keepdims=True)
    e = jnp.exp(z - zm)
    sc = e / jnp.sum(e, axis=1, keepdims=True)
    feat = jnp.sum(sc * lse1, axis=1)
    f1 = jnp.dot(feat, w1t_ref[...], preferred_element_type=jnp.float32, precision=jax.lax.Precision.HIGHEST)
    f1_ref[...] = f1
    s1 = jnp.sum(f1, axis=0)[None, :]
    s2 = jnp.sum(f1 * f1, axis=0)[None, :]
    acc = jnp.concatenate([s1, s2], axis=0)

    @pl.when(step == 0)
    def _():
        st_ref[...] = jnp.zeros_like(st_ref)

    st_ref[...] += acc


def _final_kernel(f1_ref, sc_ref, sh_ref, out_ref):
    y = f1_ref[...] * sc_ref[0][None, :] + sh_ref[0][None, :]
    out_ref[...] = jnp.where(y >= 0, y, _LEAKY * y)


def kernel(xyz, points, W0, b0, g0, be0, Wl, bl, gl, bel, Ws, W1, b1, g1, be1):
    xyz_t = xyz.transpose(0, 2, 1)
    pts_t = points.transpose(0, 2, 1)
    fps_idx, nxyz_cs = _fps_pallas(xyz)
    new_xyz = nxyz_cs.transpose(0, 2, 1)

    idx = _knn_pallas(new_xyz, xyz)

    half = _COUT // 2
    Wg = Wl[:, 3:6] + Wl[:, 6:9]
    Wm = Wl[:, 0:3] - Wl[:, 6:9]
    Wcomb = jnp.zeros((3 + _CIN, 128), jnp.float32)
    Wcomb = Wcomb.at[0:3, 0:half].set(Wg.T)
    Wcomb = Wcomb.at[3:, half:_COUT].set(W0.T)
    in2 = jnp.concatenate([xyz_t, pts_t], axis=-1)
    G = pl.pallas_call(
        _gmat_kernel,
        grid=(_B,),
        in_specs=[pl.BlockSpec((1, _N, 3 + _CIN), lambda b: (b, 0, 0)),
                  pl.BlockSpec((3 + _CIN, 128), lambda b: (0, 0))],
        out_specs=pl.BlockSpec((1, _N, 128), lambda b: (b, 0, 0)),
        out_shape=jax.ShapeDtypeStruct((_B, _N, 128), jnp.float32),
    )(in2, Wcomb)

    idx_glob = (idx + (jnp.arange(_B, dtype=jnp.int32) * _N)[:, None, None])
    Gg = _sc_gather(G.reshape(_B * _N, 128), idx_glob.reshape(-1))
    Gg = Gg.reshape(_B * _S, _K, 128)

    nxyz_flat = new_xyz.reshape(_B * _S, 3)
    Wm64 = jnp.zeros((3, _COUT), jnp.float32).at[:, 0:half].set(Wm.T)

    nsteps = (_B * _S) // _TROWS
    row_spec = pl.BlockSpec((_TROWS, _K, 128), lambda s: (s, 0, 0))
    c_spec = pl.BlockSpec((_TROWS, 3), lambda s: (s, 0))
    wm_spec = pl.BlockSpec((3, _COUT), lambda s: (0, 0))
    acc_spec = pl.BlockSpec((2, _COUT), lambda s: (0, 0))
    vec_spec = pl.BlockSpec((1, _COUT), lambda s: (0, 0))
    w64_spec = pl.BlockSpec((_COUT, _COUT), lambda s: (0, 0))

    sums = pl.pallas_call(
        _stats_kernel,
        grid=(nsteps,),
        in_specs=[row_spec, c_spec, wm_spec],
        out_specs=acc_spec,
        out_shape=jax.ShapeDtypeStruct((2, _COUT), jnp.float32),
    )(Gg, nxyz_flat, Wm64)

    n = float(_B * _S * _K)
    gf = jnp.concatenate([gl, g0])
    bef = jnp.concatenate([bel, be0])
    mean_x = sums[0] / n
    var = sums[1] / n - mean_x * mean_x
    scale64 = gf / jnp.sqrt(var + _EPS)
    shift64 = bef - mean_x * scale64

    f1, st = pl.pallas_call(
        _main_kernel,
        grid=(nsteps,),
        in_specs=[row_spec, c_spec, wm_spec, vec_spec, vec_spec,
                  w64_spec, w64_spec],
        out_specs=[pl.BlockSpec((_TROWS, _COUT), lambda s: (s, 0)), acc_spec],
        out_shape=[jax.ShapeDtypeStruct((_B * _S, _COUT), jnp.float32),
                   jax.ShapeDtypeStruct((2, _COUT), jnp.float32)],
    )(Gg, nxyz_flat, Wm64, scale64[None], shift64[None], Ws.T, W1.T)

    n1 = float(_B * _S)
    mean1 = st[0] / n1
    var1 = st[1] / n1 - mean1 * mean1
    scale1 = g1 / jnp.sqrt(var1 + _EPS)
    shift1 = be1 - mean1 * scale1

    outp = pl.pallas_call(
        _final_kernel,
        grid=(_B,),
        in_specs=[pl.BlockSpec((_S, _COUT), lambda b: (b, 0)),
                  pl.BlockSpec((1, _COUT), lambda b: (0, 0)),
                  pl.BlockSpec((1, _COUT), lambda b: (0, 0))],
        out_specs=pl.BlockSpec((_S, _COUT), lambda b: (b, 0)),
        out_shape=jax.ShapeDtypeStruct((_B * _S, _COUT), jnp.float32),
    )(f1, scale1[None], shift1[None])

    out_points = outp.reshape(_B, _S, _COUT).transpose(0, 2, 1)
    return (nxyz_cs, out_points, fps_idx)
